# Initial kernel scaffold; baseline (speedup 1.0000x reference)
import jax
import jax.numpy as jnp
from jax.experimental import pallas as pl


def kernel(x, token_table, pos_table):
    raise NotImplementedError("write your pallas kernel here")


# speedup vs baseline: 1.6742x; 1.6742x over previous
import functools

import jax
import jax.numpy as jnp
from jax import lax
from jax.experimental import pallas as pl
from jax.experimental.pallas import tpu as pltpu
from jax.experimental.pallas import tpu_sc as plsc

B = 1024
L = 200
E = 128
NC = 2
NS = 16
NW = NC * NS
ROWS_PER_W = B // NW
CH = 2
CL = L // CH

_mesh = plsc.VectorSubcoreMesh(core_axis_name="c", subcore_axis_name="s")


@functools.partial(
    pl.kernel,
    mesh=_mesh,
    out_type=jax.ShapeDtypeStruct((B, CH, CL, E), jnp.float32),
    scratch_types=[
        pltpu.VMEM((CH, CL), jnp.int32),
        pltpu.VMEM((L, E), jnp.float32),
        pltpu.VMEM((CH, CL, E), jnp.float32),
        pltpu.SemaphoreType.DMA,
    ],
)
def _emb_kernel(x_hbm, tok_hbm, pos_hbm, out_hbm, idx_v, pos_v, rows_v, sem):
    wid = lax.axis_index("s") * NC + lax.axis_index("c")
    pltpu.sync_copy(pos_hbm, pos_v)

    def row_body(i, carry):
        b = wid * ROWS_PER_W + i
        pltpu.sync_copy(x_hbm.at[b], idx_v)
        cps = [
            pltpu.async_copy(tok_hbm.at[idx_v.at[c]], rows_v.at[c], sem)
            for c in range(CH)
        ]
        for cp in cps:
            cp.wait()
        for c in range(CH):
            def add_body(r, _):
                for j in range(E // 16):
                    sl = pl.ds(j * 16, 16)
                    rows_v[c, r, sl] = rows_v[c, r, sl] + pos_v[c * CL + r, sl]
                return 0
            lax.fori_loop(0, CL, add_body, 0)
        pltpu.sync_copy(rows_v, out_hbm.at[b])
        return carry

    lax.fori_loop(0, ROWS_PER_W, row_body, 0)


BBLK = 8


def _mask_body(x_ref, o_ref):
    m = (x_ref[...] != 0).astype(jnp.int32)
    o_ref[...] = m[:, :, None] * m[:, None, :]


def kernel(x, token_table, pos_table):
    x_sc = x.reshape(B, CH, CL)
    out = _emb_kernel(x_sc, token_table, pos_table).reshape(B, L, E)
    attn = pl.pallas_call(
        _mask_body,
        grid=(B // BBLK,),
        in_specs=[pl.BlockSpec((BBLK, L), lambda i: (i, 0))],
        out_specs=pl.BlockSpec((BBLK, L, L), lambda i: (i, 0, 0)),
        out_shape=jax.ShapeDtypeStruct((B, L, L), jnp.int32),
    )(x)
    return out, attn.reshape(B, 1, L, L)


# speedup vs baseline: 2.3192x; 1.3853x over previous
import functools

import jax
import jax.numpy as jnp
from jax import lax
from jax.experimental import pallas as pl
from jax.experimental.pallas import tpu as pltpu
from jax.experimental.pallas import tpu_sc as plsc

B = 1024
L = 200
E = 128
NC = 2
NS = 16
NW = NC * NS
ROWS_PER_W = B // NW
CH = 2
CL = L // CH

_mesh = plsc.VectorSubcoreMesh(core_axis_name="c", subcore_axis_name="s")


@functools.partial(
    pl.kernel,
    mesh=_mesh,
    out_type=jax.ShapeDtypeStruct((B, L, E), jnp.float32),
    scratch_types=[
        pltpu.VMEM((CH, CL), jnp.int32),
        pltpu.VMEM((L, E), jnp.float32),
        pltpu.VMEM((L, E), jnp.float32),
        pltpu.SemaphoreType.DMA,
    ],
)
def _emb_kernel(x_hbm, tok_hbm, pos_hbm, out_hbm, idx_v, pos_v, rows_v, sem):
    wid = lax.axis_index("s") * NC + lax.axis_index("c")
    pltpu.sync_copy(pos_hbm, pos_v)

    def row_body(i, carry):
        b = wid * ROWS_PER_W + i
        pltpu.sync_copy(x_hbm.at[b], idx_v)
        cps = [
            pltpu.async_copy(
                tok_hbm.at[idx_v.at[c]], rows_v.at[pl.ds(c * CL, CL)], sem
            )
            for c in range(CH)
        ]
        for cp in cps:
            cp.wait()

        def add_body(r, _):
            for j in range(E // 16):
                sl = pl.ds(j * 16, 16)
                rows_v[r, sl] = rows_v[r, sl] + pos_v[r, sl]
            return 0

        lax.fori_loop(0, L, add_body, 0)
        pltpu.sync_copy(rows_v, out_hbm.at[b])
        return carry

    lax.fori_loop(0, ROWS_PER_W, row_body, 0)


BBLK = 8


def _mask_body(x_ref, o_ref):
    m = (x_ref[...] != 0).astype(jnp.int32)
    o_ref[...] = m[:, None, :, None] * m[:, None, None, :]


def kernel(x, token_table, pos_table):
    x_sc = x.reshape(B, CH, CL)
    out = _emb_kernel(x_sc, token_table, pos_table)
    attn_mask = pl.pallas_call(
        _mask_body,
        grid=(B // BBLK,),
        in_specs=[pl.BlockSpec((BBLK, L), lambda i: (i, 0))],
        out_specs=pl.BlockSpec((BBLK, 1, L, L), lambda i: (i, 0, 0, 0)),
        out_shape=jax.ShapeDtypeStruct((B, 1, L, L), jnp.int32),
    )(x)
    return out, attn_mask
